# Initial kernel scaffold; baseline (speedup 1.0000x reference)
import jax
import jax.numpy as jnp
from jax.experimental import pallas as pl


def kernel(x, W1, b1, W2, b2, Wg):
    raise NotImplementedError("write your pallas kernel here")


# speedup vs baseline: 1.6983x; 1.6983x over previous
import jax
import jax.numpy as jnp
from jax.experimental import pallas as pl

_SQRT_2_OVER_PI = 0.7978845608028654


def _mlp_body(x_ref, w1_ref, b1_ref, w2_ref, b2_ref, o_ref):
    j = pl.program_id(1)
    h = jax.lax.dot_general(
        x_ref[...], w1_ref[...], (((1,), (1,)), ((), ())),
        preferred_element_type=jnp.float32)
    h = h + b1_ref[...]
    h = 0.5 * h * (1.0 + jnp.tanh(_SQRT_2_OVER_PI * (h + 0.044715 * h * h * h)))
    p = jax.lax.dot_general(
        h, w2_ref[...], (((1,), (1,)), ((), ())),
        preferred_element_type=jnp.float32)

    @pl.when(j == 0)
    def _():
        o_ref[...] = p + b2_ref[...]

    @pl.when(j > 0)
    def _():
        o_ref[...] += p


def kernel(x, W1, b1, W2, b2, Wg):
    B, S, D = x.shape
    M = B * S
    FF = W1.shape[0]
    xf = x.reshape(M, D)
    BM = min(256, M)
    BF = min(1024, FF)
    grid = (M // BM, FF // BF)
    out = pl.pallas_call(
        _mlp_body,
        grid=grid,
        in_specs=[
            pl.BlockSpec((BM, D), lambda i, j: (i, 0)),
            pl.BlockSpec((BF, D), lambda i, j: (j, 0)),
            pl.BlockSpec((1, BF), lambda i, j: (0, j)),
            pl.BlockSpec((D, BF), lambda i, j: (0, j)),
            pl.BlockSpec((1, D), lambda i, j: (0, 0)),
        ],
        out_specs=pl.BlockSpec((BM, D), lambda i, j: (i, 0)),
        out_shape=jax.ShapeDtypeStruct((M, D), jnp.float32),
    )(xf, W1, b1.reshape(1, FF), W2, b2.reshape(1, D))
    return out.reshape(B, S, D)


# speedup vs baseline: 2.9437x; 1.7333x over previous
import jax
import jax.numpy as jnp
from jax.experimental import pallas as pl

_SQRT_2_OVER_PI = 0.7978845608028654


def _mlp_body(x_ref, w1_ref, b1_ref, w2_ref, b2_ref, o_ref):
    j = pl.program_id(1)
    h = jax.lax.dot_general(
        x_ref[...], w1_ref[...], (((1,), (1,)), ((), ())),
        preferred_element_type=jnp.float32)
    h = h + b1_ref[...]
    h = 0.5 * h * (1.0 + jnp.tanh(_SQRT_2_OVER_PI * (h + 0.044715 * h * h * h)))
    p = jax.lax.dot_general(
        h, w2_ref[...], (((1,), (1,)), ((), ())),
        preferred_element_type=jnp.float32)

    @pl.when(j == 0)
    def _():
        o_ref[...] = p + b2_ref[...]

    @pl.when(j > 0)
    def _():
        o_ref[...] += p


def kernel(x, W1, b1, W2, b2, Wg):
    B, S, D = x.shape
    M = B * S
    FF = W1.shape[0]
    xf = x.reshape(M, D)
    BM = min(512, M)
    BF = min(1024, FF)
    grid = (M // BM, FF // BF)
    out = pl.pallas_call(
        _mlp_body,
        grid=grid,
        in_specs=[
            pl.BlockSpec((BM, D), lambda i, j: (i, 0)),
            pl.BlockSpec((BF, D), lambda i, j: (j, 0)),
            pl.BlockSpec((1, BF), lambda i, j: (0, j)),
            pl.BlockSpec((D, BF), lambda i, j: (0, j)),
            pl.BlockSpec((1, D), lambda i, j: (0, 0)),
        ],
        out_specs=pl.BlockSpec((BM, D), lambda i, j: (i, 0)),
        out_shape=jax.ShapeDtypeStruct((M, D), jnp.float32),
    )(xf, W1, b1.reshape(1, FF), W2, b2.reshape(1, D))
    return out.reshape(B, S, D)


# speedup vs baseline: 2.9975x; 1.0183x over previous
import jax
import jax.numpy as jnp
from jax.experimental import pallas as pl

_SQRT_2_OVER_PI = 0.7978845608028654


def _mlp_body(x_ref, w1_ref, b1_ref, w2_ref, b2_ref, o_ref):
    j = pl.program_id(1)
    h = jax.lax.dot_general(
        x_ref[...], w1_ref[...], (((1,), (1,)), ((), ())),
        preferred_element_type=jnp.float32)
    h = h + b1_ref[...]
    h = 0.5 * h * (1.0 + jnp.tanh(_SQRT_2_OVER_PI * (h + 0.044715 * h * h * h)))
    p = jax.lax.dot_general(
        h, w2_ref[...], (((1,), (1,)), ((), ())),
        preferred_element_type=jnp.float32)

    @pl.when(j == 0)
    def _():
        o_ref[...] = p + b2_ref[...]

    @pl.when(j > 0)
    def _():
        o_ref[...] += p


def kernel(x, W1, b1, W2, b2, Wg):
    B, S, D = x.shape
    M = B * S
    FF = W1.shape[0]
    xf = x.reshape(M, D)
    BM = min(1024, M)
    BF = min(512, FF)
    grid = (M // BM, FF // BF)
    out = pl.pallas_call(
        _mlp_body,
        grid=grid,
        in_specs=[
            pl.BlockSpec((BM, D), lambda i, j: (i, 0)),
            pl.BlockSpec((BF, D), lambda i, j: (j, 0)),
            pl.BlockSpec((1, BF), lambda i, j: (0, j)),
            pl.BlockSpec((D, BF), lambda i, j: (0, j)),
            pl.BlockSpec((1, D), lambda i, j: (0, 0)),
        ],
        out_specs=pl.BlockSpec((BM, D), lambda i, j: (i, 0)),
        out_shape=jax.ShapeDtypeStruct((M, D), jnp.float32),
    )(xf, W1, b1.reshape(1, FF), W2, b2.reshape(1, D))
    return out.reshape(B, S, D)
